# baseline (speedup 1.0000x reference)
import functools

import jax
import jax.numpy as jnp
from jax import lax
from jax.experimental import pallas as pl
from jax.experimental.pallas import tpu as pltpu
from jax.experimental.pallas import tpu_sc as plsc

V = 1000000
D = 64
B = 16384
C = 4
K = 5
NT = 1 + K

_info = plsc.get_sparse_core_info()
NC = _info.num_cores
NS = _info.num_subcores
L = _info.num_lanes
NW = NC * NS
B_PER_W = B // NW
NCHUNK = 64
N_CHUNKS = B_PER_W // NCHUNK

NBLK = (V + 127) // 128
WIN = 2
NFULLWIN = 3904
WPW = NFULLWIN // NW
SCR_ROWS = NBLK * 64

_SC_PARAMS = pltpu.CompilerParams(
    needs_layout_passes=False, use_tc_tiling_on_sc=True)


def _transpose_win(st, ov, iota, ngroups=16):
    for g in range(ngroups):
        cvec = iota + (16 * g)
        rows = lax.shift_right_logical(cvec, 1)
        colbase = (cvec & 1) * 64

        def body(d, carry):
            for u in range(4):
                v = st[d * 4 + u, pl.ds(16 * g, 16)]
                plsc.store_scatter(ov, [rows, colbase + (d * 4 + u)], v)
            return carry

        lax.fori_loop(0, 16, body, 0)


def _sc_transpose_kernel(tA, tB, scrA, scrB,
                         st0, st1, ov0, ov1, sr0, sr1, sw0, sw1):
    wid = lax.axis_index("s") * NC + lax.axis_index("c")
    iota = jnp.arange(L, dtype=jnp.int32)

    def rd(tbl, win, st, sem):
        return pltpu.make_async_copy(
            tbl.at[:, pl.ds(pl.multiple_of(win * 256, 256), 256)], st, sem)

    def wr(scr, win, ov, sem):
        return pltpu.make_async_copy(
            ov, scr.at[pl.ds(pl.multiple_of(win * 128, 128), 128)], sem)

    for tbl, scr in ((tA, scrA), (tB, scrB)):
        w0 = wid * WPW
        rd(tbl, w0, st0, sr0).start()

        def pair_body(i, carry):
            k0 = w0 + 2 * i
            rd(tbl, k0 + 1, st1, sr1).start()
            rd(tbl, k0, st0, sr0).wait()

            @pl.when(i > 0)
            def _():
                wr(scr, k0 - 2, ov0, sw0).wait()
            _transpose_win(st0, ov0, iota)
            wr(scr, k0, ov0, sw0).start()
            @pl.when(i < (WPW // 2) - 1)
            def _():
                rd(tbl, k0 + 2, st0, sr0).start()
            rd(tbl, k0 + 1, st1, sr1).wait()

            @pl.when(i > 0)
            def _():
                wr(scr, k0 - 1, ov1, sw1).wait()
            _transpose_win(st1, ov1, iota)
            wr(scr, k0 + 1, ov1, sw1).start()
            return carry

        lax.fori_loop(0, WPW // 2, pair_body, 0)
        wr(scr, w0 + WPW - 2, ov0, sw0).wait()
        wr(scr, w0 + WPW - 1, ov1, sw1).wait()

        @pl.when(wid < 2)
        def _():
            win = NFULLWIN + wid
            pltpu.sync_copy(
                tbl.at[:, pl.ds(pl.multiple_of(win * 256, 256), 256)], st0)
            _transpose_win(st0, ov0, iota)
            pltpu.sync_copy(
                ov0, scr.at[pl.ds(pl.multiple_of(win * 128, 128), 128)])

        @pl.when(wid == 2)
        def _():
            c0 = pl.multiple_of((NBLK - 1) * 128 + 0 * wid, 128)
            pltpu.sync_copy(tbl.at[:, pl.ds(c0, 128)], st0.at[:, pl.ds(0, 128)])
            _transpose_win(st0, ov0, iota)
            pltpu.sync_copy(ov0.at[pl.ds(0, 64)],
                            scr.at[pl.ds((NBLK - 1) * 64, 64)])


@jax.jit
def _sc_transpose(tA, tB):
    mesh = plsc.VectorSubcoreMesh(core_axis_name="c", subcore_axis_name="s")
    f = functools.partial(
        pl.kernel, mesh=mesh,
        out_type=(jax.ShapeDtypeStruct((SCR_ROWS, 128), jnp.float32),
                  jax.ShapeDtypeStruct((SCR_ROWS, 128), jnp.float32)),
        scratch_types=[
            pltpu.VMEM((64, 256), jnp.float32),
            pltpu.VMEM((64, 256), jnp.float32),
            pltpu.VMEM((128, 128), jnp.float32),
            pltpu.VMEM((128, 128), jnp.float32),
            pltpu.SemaphoreType.DMA,
            pltpu.SemaphoreType.DMA,
            pltpu.SemaphoreType.DMA,
            pltpu.SemaphoreType.DMA,
        ],
        compiler_params=_SC_PARAMS,
    )(_sc_transpose_kernel)
    return f(tA, tB)


def _sc_scores_kernel(cpair_hbm, coff_hbm, opair_hbm, ooff_hbm, scrA, scrB,
                      scores_hbm,
                      cpair_v, coff_v, opair_v, ooff_v,
                      ctx_rows_v, out_rows_v, cm_t_v, scores_v, sem_c, sem_o):
    wid = lax.axis_index("s") * NC + lax.axis_index("c")
    iota = jnp.arange(L, dtype=jnp.int32)
    pltpu.sync_copy(cpair_hbm.at[pl.ds(wid * (B_PER_W * C), B_PER_W * C)],
                    cpair_v)
    pltpu.sync_copy(coff_hbm.at[pl.ds(wid * (B_PER_W * C), B_PER_W * C)],
                    coff_v.at[pl.ds(0, B_PER_W * C)])
    pltpu.sync_copy(opair_hbm.at[pl.ds(wid * (B_PER_W * NT), B_PER_W * NT)],
                    opair_v)
    pltpu.sync_copy(ooff_hbm.at[pl.ds(wid * (B_PER_W * NT), B_PER_W * NT)],
                    ooff_v)
    for chunk in range(N_CHUNKS):
        lc = chunk * NCHUNK * C
        lo = chunk * NCHUNK * NT
        cps = []
        for g in range(NCHUNK * C // 128):
            cps.append(pltpu.async_copy(
                scrA.at[cpair_v.at[pl.ds(lc + g * 128, 128)]],
                ctx_rows_v.at[pl.ds(g * 128, 128)], sem_c))
        for g in range(NCHUNK * NT // 128):
            cps.append(pltpu.async_copy(
                scrB.at[opair_v.at[pl.ds(lo + g * 128, 128)]],
                out_rows_v.at[pl.ds(g * 128, 128)], sem_o))
        for cp in cps:
            cp.wait()

        def mean_body(b, carry):
            r0 = C * b
            colb = jnp.full((L,), b, dtype=jnp.int32)
            offs_vec = coff_v[pl.ds(lc + r0, L)]
            offs = [offs_vec[k] for k in range(C)]
            for m in range(D // L):
                v = ctx_rows_v[r0, pl.ds(offs[0] + m * L, L)]
                for k in range(1, C):
                    v = v + ctx_rows_v[r0 + k, pl.ds(offs[k] + m * L, L)]
                plsc.store_scatter(cm_t_v, [iota + (m * L), colb],
                                   v * (1.0 / C))
            return carry

        lax.fori_loop(0, NCHUNK, mean_body, 0)

        for g in range(NCHUNK // L):
            b0 = g * L
            rows = [(iota + b0) * NT + t for t in range(NT)]
            offs = [plsc.load_gather(ooff_v, [rows[t] + lo])
                    for t in range(NT)]

            def dot_body(d, accs):
                cm = cm_t_v[d, pl.ds(b0, L)]
                return tuple(
                    accs[t] + cm * plsc.load_gather(
                        out_rows_v, [rows[t], offs[t] + d])
                    for t in range(NT))

            accs = lax.fori_loop(
                0, D, dot_body,
                tuple(jnp.zeros((L,), jnp.float32) for _ in range(NT)))
            for t in range(NT):
                scores_v[t, pl.ds(chunk * NCHUNK + b0, L)] = accs[t]

    pltpu.sync_copy(scores_v,
                    scores_hbm.at[:, pl.ds(wid * B_PER_W, B_PER_W)])


@jax.jit
def _sc_scores(cpair, coff, opair, ooff, scrA, scrB):
    mesh = plsc.VectorSubcoreMesh(core_axis_name="c", subcore_axis_name="s")
    f = functools.partial(
        pl.kernel, mesh=mesh,
        out_type=jax.ShapeDtypeStruct((8, B), jnp.float32),
        scratch_types=[
            pltpu.VMEM((B_PER_W * C,), jnp.int32),
            pltpu.VMEM((B_PER_W * C + L,), jnp.int32),
            pltpu.VMEM((B_PER_W * NT,), jnp.int32),
            pltpu.VMEM((B_PER_W * NT,), jnp.int32),
            pltpu.VMEM((NCHUNK * C, 128), jnp.float32),
            pltpu.VMEM((NCHUNK * NT, 128), jnp.float32),
            pltpu.VMEM((D, NCHUNK), jnp.float32),
            pltpu.VMEM((8, B_PER_W), jnp.float32),
            pltpu.SemaphoreType.DMA,
            pltpu.SemaphoreType.DMA,
        ],
        compiler_params=_SC_PARAMS,
    )(_sc_scores_kernel)
    return f(cpair, coff, opair, ooff, scrA, scrB)


def _loss_body(s_ref, o_ref):
    x = s_ref[...]
    rowid = lax.broadcasted_iota(jnp.int32, x.shape, 0)
    y = jnp.where(rowid == 0, -x, x)
    sp = jnp.maximum(y, 0.0) + jnp.log(1.0 + jnp.exp(-jnp.abs(y)))
    sp = jnp.where(rowid < NT, sp, 0.0)
    o_ref[0, 0] = jnp.sum(sp) * (1.0 / B)


@jax.jit
def _tc_loss(scores):
    return pl.pallas_call(
        _loss_body,
        out_shape=jax.ShapeDtypeStruct((1, 1), jnp.float32),
        out_specs=pl.BlockSpec(memory_space=pltpu.SMEM),
    )(scores)


def kernel(center, context, neg_context, in_embed, out_embed):
    ci = context.astype(jnp.int32).reshape(B * C)
    oi = jnp.concatenate(
        [center.astype(jnp.int32), neg_context.astype(jnp.int32)],
        axis=1).reshape(B * NT)
    cpair = ci >> 1
    coff = (ci & 1) * D
    opair = oi >> 1
    ooff = (oi & 1) * D
    tA = jnp.swapaxes(in_embed, 0, 1)
    tB = jnp.swapaxes(out_embed, 0, 1)
    scrA, scrB = _sc_transpose(tA, tB)
    scores = _sc_scores(cpair, coff, opair, ooff, scrA, scrB)
    loss = _tc_loss(scores)
    return loss[0, 0]


# speedup vs baseline: 2.0515x; 2.0515x over previous
import functools

import jax
import jax.numpy as jnp
from jax import lax
from jax.experimental import pallas as pl
from jax.experimental.pallas import tpu as pltpu
from jax.experimental.pallas import tpu_sc as plsc

V = 1000000
D = 64
B = 16384
C = 4
K = 5
NT = 1 + K

_info = plsc.get_sparse_core_info()
NC = _info.num_cores
NS = _info.num_subcores
L = _info.num_lanes
NW = NC * NS
B_PER_W = B // NW
NCHUNK = 64
N_CHUNKS = B_PER_W // NCHUNK

NBLK = (V + 127) // 128
WIN = 2
NFULLWIN = 3904
WPW = NFULLWIN // NW
SCR_ROWS = NBLK * 64

_SC_PARAMS = pltpu.CompilerParams(
    needs_layout_passes=False, use_tc_tiling_on_sc=True)


def _sc_scores_kernel(cpair_hbm, coff_hbm, opair_hbm, ooff_hbm, scrA, scrB,
                      scores_hbm,
                      cpair_v, coff_v, opair_v, ooff_v,
                      ctx_rows_v, out_rows_v, cm_t_v, scores_v, sem_c, sem_o):
    wid = lax.axis_index("s") * NC + lax.axis_index("c")
    iota = jnp.arange(L, dtype=jnp.int32)
    pltpu.sync_copy(cpair_hbm.at[pl.ds(wid * (B_PER_W * C), B_PER_W * C)],
                    cpair_v)
    pltpu.sync_copy(coff_hbm.at[pl.ds(wid * (B_PER_W * C), B_PER_W * C)],
                    coff_v.at[pl.ds(0, B_PER_W * C)])
    pltpu.sync_copy(opair_hbm.at[pl.ds(wid * (B_PER_W * NT), B_PER_W * NT)],
                    opair_v)
    pltpu.sync_copy(ooff_hbm.at[pl.ds(wid * (B_PER_W * NT), B_PER_W * NT)],
                    ooff_v)
    for chunk in range(N_CHUNKS):
        lc = chunk * NCHUNK * C
        lo = chunk * NCHUNK * NT
        cps = []
        for g in range(NCHUNK * C // 128):
            cps.append(pltpu.async_copy(
                scrA.at[cpair_v.at[pl.ds(lc + g * 128, 128)]],
                ctx_rows_v.at[pl.ds(g * 128, 128)], sem_c))
        for g in range(NCHUNK * NT // 128):
            cps.append(pltpu.async_copy(
                scrB.at[opair_v.at[pl.ds(lo + g * 128, 128)]],
                out_rows_v.at[pl.ds(g * 128, 128)], sem_o))
        for cp in cps:
            cp.wait()

        def mean_body(b, carry):
            r0 = C * b
            colb = jnp.full((L,), b, dtype=jnp.int32)
            offs_vec = coff_v[pl.ds(lc + r0, L)]
            offs = [offs_vec[k] for k in range(C)]
            for m in range(D // L):
                v = ctx_rows_v[r0, pl.ds(offs[0] + m * L, L)]
                for k in range(1, C):
                    v = v + ctx_rows_v[r0 + k, pl.ds(offs[k] + m * L, L)]
                plsc.store_scatter(cm_t_v, [iota + (m * L), colb],
                                   v * (1.0 / C))
            return carry

        lax.fori_loop(0, NCHUNK, mean_body, 0)

        for g in range(NCHUNK // L):
            b0 = g * L
            rows = [(iota + b0) * NT + t for t in range(NT)]
            offs = [plsc.load_gather(ooff_v, [rows[t] + lo])
                    for t in range(NT)]

            def dot_body(d, accs):
                cm = cm_t_v[d, pl.ds(b0, L)]
                return tuple(
                    accs[t] + cm * plsc.load_gather(
                        out_rows_v, [rows[t], offs[t] + d])
                    for t in range(NT))

            accs = lax.fori_loop(
                0, D, dot_body,
                tuple(jnp.zeros((L,), jnp.float32) for _ in range(NT)))
            for t in range(NT):
                scores_v[t, pl.ds(chunk * NCHUNK + b0, L)] = accs[t]

    pltpu.sync_copy(scores_v,
                    scores_hbm.at[:, pl.ds(wid * B_PER_W, B_PER_W)])


@jax.jit
def _sc_scores(cpair, coff, opair, ooff, scrA, scrB):
    mesh = plsc.VectorSubcoreMesh(core_axis_name="c", subcore_axis_name="s")
    f = functools.partial(
        pl.kernel, mesh=mesh,
        out_type=jax.ShapeDtypeStruct((8, B), jnp.float32),
        scratch_types=[
            pltpu.VMEM((B_PER_W * C,), jnp.int32),
            pltpu.VMEM((B_PER_W * C + L,), jnp.int32),
            pltpu.VMEM((B_PER_W * NT,), jnp.int32),
            pltpu.VMEM((B_PER_W * NT,), jnp.int32),
            pltpu.VMEM((NCHUNK * C, 128), jnp.float32),
            pltpu.VMEM((NCHUNK * NT, 128), jnp.float32),
            pltpu.VMEM((D, NCHUNK), jnp.float32),
            pltpu.VMEM((8, B_PER_W), jnp.float32),
            pltpu.SemaphoreType.DMA,
            pltpu.SemaphoreType.DMA,
        ],
        compiler_params=_SC_PARAMS,
    )(_sc_scores_kernel)
    return f(cpair, coff, opair, ooff, scrA, scrB)


def _loss_body(s_ref, o_ref):
    x = s_ref[...]
    rowid = lax.broadcasted_iota(jnp.int32, x.shape, 0)
    y = jnp.where(rowid == 0, -x, x)
    sp = jnp.maximum(y, 0.0) + jnp.log(1.0 + jnp.exp(-jnp.abs(y)))
    sp = jnp.where(rowid < NT, sp, 0.0)
    o_ref[0, 0] = jnp.sum(sp) * (1.0 / B)


@jax.jit
def _tc_loss(scores):
    return pl.pallas_call(
        _loss_body,
        out_shape=jax.ShapeDtypeStruct((1, 1), jnp.float32),
        out_specs=pl.BlockSpec(memory_space=pltpu.SMEM),
    )(scores)


def kernel(center, context, neg_context, in_embed, out_embed):
    ci = context.astype(jnp.int32).reshape(B * C)
    oi = jnp.concatenate(
        [center.astype(jnp.int32), neg_context.astype(jnp.int32)],
        axis=1).reshape(B * NT)
    cpair = ci >> 1
    coff = (ci & 1) * D
    opair = oi >> 1
    ooff = (oi & 1) * D
    scrA = jnp.reshape(in_embed, (V // 2, 2 * D))
    scrB = jnp.reshape(out_embed, (V // 2, 2 * D))
    scores = _sc_scores(cpair, coff, opair, ooff, scrA, scrB)
    loss = _tc_loss(scores)
    return loss[0, 0]


# speedup vs baseline: 2.1271x; 1.0369x over previous
import functools

import jax
import jax.numpy as jnp
from jax import lax
from jax.experimental import pallas as pl
from jax.experimental.pallas import tpu as pltpu
from jax.experimental.pallas import tpu_sc as plsc

V = 1000000
D = 64
B = 16384
C = 4
K = 5
NT = 1 + K

_info = plsc.get_sparse_core_info()
NC = _info.num_cores
NS = _info.num_subcores
L = _info.num_lanes
NW = NC * NS
B_PER_W = B // NW
NCHUNK = 32
N_CHUNKS = B_PER_W // NCHUNK


def _sc_scores_kernel(ctx_idx_hbm, out_idx_hbm, in_embed_hbm, out_embed_hbm,
                      scores_hbm,
                      ctx_idx_v, out_idx_v, ctx_rows0, ctx_rows1,
                      out_rows0, out_rows1, cm_t_v, scores_v, sem_c0, sem_c1,
                      sem_o0, sem_o1):
    wid = lax.axis_index("s") * NC + lax.axis_index("c")
    iota = jnp.arange(L, dtype=jnp.int32)
    pltpu.sync_copy(ctx_idx_hbm.at[pl.ds(wid * (B_PER_W * C), B_PER_W * C)],
                    ctx_idx_v)
    pltpu.sync_copy(out_idx_hbm.at[pl.ds(wid * (B_PER_W * NT), B_PER_W * NT)],
                    out_idx_v)

    ctx_rows = (ctx_rows0, ctx_rows1)
    out_rows = (out_rows0, out_rows1)
    sems_c = (sem_c0, sem_c1)
    sems_o = (sem_o0, sem_o1)

    def fire(chunk):
        p = chunk % 2
        lc = chunk * NCHUNK * C
        lo = chunk * NCHUNK * NT
        cps = [pltpu.async_copy(
            in_embed_hbm.at[ctx_idx_v.at[pl.ds(lc, NCHUNK * C)]],
            ctx_rows[p], sems_c[p])]
        cps.append(pltpu.async_copy(
            out_embed_hbm.at[out_idx_v.at[pl.ds(lo, 128)]],
            out_rows[p].at[pl.ds(0, 128)], sems_o[p]))
        cps.append(pltpu.async_copy(
            out_embed_hbm.at[out_idx_v.at[pl.ds(lo + 128, NCHUNK * NT - 128)]],
            out_rows[p].at[pl.ds(128, NCHUNK * NT - 128)], sems_o[p]))
        return cps

    cps_cur = fire(0)
    for chunk in range(N_CHUNKS):
        p = chunk % 2
        cps_next = fire(chunk + 1) if chunk + 1 < N_CHUNKS else []
        for cp in cps_cur:
            cp.wait()
        cps_cur = cps_next
        ctx_rows_v = ctx_rows[p]
        out_rows_v = out_rows[p]

        def mean_body(b, carry):
            r0 = C * b
            colb = jnp.full((L,), b, dtype=jnp.int32)
            for m in range(D // L):
                s = pl.ds(m * L, L)
                v = (ctx_rows_v[r0, s] + ctx_rows_v[r0 + 1, s]
                     + ctx_rows_v[r0 + 2, s] + ctx_rows_v[r0 + 3, s])
                plsc.store_scatter(cm_t_v, [iota + (m * L), colb],
                                   v * (1.0 / C))
            return carry

        lax.fori_loop(0, NCHUNK, mean_body, 0)

        for g in range(NCHUNK // L):
            b0 = g * L
            rows = [(iota + b0) * NT + t for t in range(NT)]

            def dot_body(d, accs):
                cm = cm_t_v[d, pl.ds(b0, L)]
                cold = jnp.full((L,), d, dtype=jnp.int32)
                return tuple(
                    accs[t] + cm * plsc.load_gather(out_rows_v, [rows[t], cold])
                    for t in range(NT))

            accs = lax.fori_loop(
                0, D, dot_body,
                tuple(jnp.zeros((L,), jnp.float32) for _ in range(NT)))
            for t in range(NT):
                scores_v[t, pl.ds(chunk * NCHUNK + b0, L)] = accs[t]

    for t in range(NT):
        pltpu.sync_copy(scores_v.at[t],
                        scores_hbm.at[t, pl.ds(wid * B_PER_W, B_PER_W)])


@jax.jit
def _sc_scores(ctx_idx, out_idx, in_embed, out_embed):
    mesh = plsc.VectorSubcoreMesh(core_axis_name="c", subcore_axis_name="s")
    f = functools.partial(
        pl.kernel, mesh=mesh,
        out_type=jax.ShapeDtypeStruct((NT, B), jnp.float32),
        scratch_types=[
            pltpu.VMEM((B_PER_W * C,), jnp.int32),
            pltpu.VMEM((B_PER_W * NT,), jnp.int32),
            pltpu.VMEM((NCHUNK * C, D), jnp.float32),
            pltpu.VMEM((NCHUNK * C, D), jnp.float32),
            pltpu.VMEM((NCHUNK * NT, D), jnp.float32),
            pltpu.VMEM((NCHUNK * NT, D), jnp.float32),
            pltpu.VMEM((D, NCHUNK), jnp.float32),
            pltpu.VMEM((NT, B_PER_W), jnp.float32),
            pltpu.SemaphoreType.DMA,
            pltpu.SemaphoreType.DMA,
            pltpu.SemaphoreType.DMA,
            pltpu.SemaphoreType.DMA,
        ],
        compiler_params=pltpu.CompilerParams(
            needs_layout_passes=False, use_tc_tiling_on_sc=False),
    )(_sc_scores_kernel)
    return f(ctx_idx, out_idx, in_embed, out_embed)


def _loss_body(s_ref, o_ref):
    x = s_ref[...]
    is_pos = lax.broadcasted_iota(jnp.int32, x.shape, 0) == 0
    y = jnp.where(is_pos, -x, x)
    sp = jnp.maximum(y, 0.0) + jnp.log(1.0 + jnp.exp(-jnp.abs(y)))
    o_ref[0, 0] = jnp.sum(sp) * (1.0 / B)


@jax.jit
def _tc_loss(scores):
    return pl.pallas_call(
        _loss_body,
        out_shape=jax.ShapeDtypeStruct((1, 1), jnp.float32),
        out_specs=pl.BlockSpec(memory_space=pltpu.SMEM),
    )(scores)


def kernel(center, context, neg_context, in_embed, out_embed):
    ctx_idx = context.astype(jnp.int32).reshape(B * C)
    out_idx = jnp.concatenate(
        [center.astype(jnp.int32), neg_context.astype(jnp.int32)],
        axis=1).reshape(B * NT)
    scores = _sc_scores(ctx_idx, out_idx, in_embed, out_embed)
    loss = _tc_loss(scores)
    return loss[0, 0]


# speedup vs baseline: 2.1310x; 1.0018x over previous
import functools

import jax
import jax.numpy as jnp
from jax import lax
from jax.experimental import pallas as pl
from jax.experimental.pallas import tpu as pltpu
from jax.experimental.pallas import tpu_sc as plsc

V = 1000000
D = 64
B = 16384
C = 4
K = 5
NT = 1 + K

_info = plsc.get_sparse_core_info()
NC = _info.num_cores
NS = _info.num_subcores
L = _info.num_lanes
NW = NC * NS
B_PER_W = B // NW
NCHUNK = 32
N_CHUNKS = B_PER_W // NCHUNK


def _sc_scores_kernel(ctx_idx_hbm, out_idx_hbm, in_embed_hbm, out_embed_hbm,
                      scores_hbm,
                      ctx_idx_v, out_idx_v, ctx_rows0, ctx_rows1,
                      out_rows0, out_rows1, cm_t_v, scores_v, sem_c0, sem_c1,
                      sem_o0, sem_o1):
    wid = lax.axis_index("s") * NC + lax.axis_index("c")
    iota = jnp.arange(L, dtype=jnp.int32)
    pltpu.sync_copy(ctx_idx_hbm.at[pl.ds(wid * (B_PER_W * C), B_PER_W * C)],
                    ctx_idx_v)
    pltpu.sync_copy(out_idx_hbm.at[pl.ds(wid * (B_PER_W * NT), B_PER_W * NT)],
                    out_idx_v)

    ctx_rows = (ctx_rows0, ctx_rows1)
    out_rows = (out_rows0, out_rows1)
    sems_c = (sem_c0, sem_c1)
    sems_o = (sem_o0, sem_o1)

    def fire(chunk):
        p = chunk % 2
        lc = chunk * NCHUNK * C
        lo = chunk * NCHUNK * NT
        cps = [pltpu.async_copy(
            in_embed_hbm.at[ctx_idx_v.at[pl.ds(lc, NCHUNK * C)]],
            ctx_rows[p], sems_c[p])]
        cps.append(pltpu.async_copy(
            out_embed_hbm.at[out_idx_v.at[pl.ds(lo, 128)]],
            out_rows[p].at[pl.ds(0, 128)], sems_o[p]))
        cps.append(pltpu.async_copy(
            out_embed_hbm.at[out_idx_v.at[pl.ds(lo + 128, NCHUNK * NT - 128)]],
            out_rows[p].at[pl.ds(128, NCHUNK * NT - 128)], sems_o[p]))
        return cps

    cps_cur = fire(0)
    for chunk in range(N_CHUNKS):
        p = chunk % 2
        cps_next = fire(chunk + 1) if chunk + 1 < N_CHUNKS else []
        cps_cur[0].wait()
        ctx_rows_v = ctx_rows[p]
        out_rows_v = out_rows[p]

        def mean_body(b, carry):
            r0 = C * b
            colb = jnp.full((L,), b, dtype=jnp.int32)
            for m in range(D // L):
                s = pl.ds(m * L, L)
                v = (ctx_rows_v[r0, s] + ctx_rows_v[r0 + 1, s]
                     + ctx_rows_v[r0 + 2, s] + ctx_rows_v[r0 + 3, s])
                plsc.store_scatter(cm_t_v, [iota + (m * L), colb],
                                   v * (1.0 / C))
            return carry

        lax.fori_loop(0, NCHUNK, mean_body, 0)
        for cp in cps_cur[1:]:
            cp.wait()
        cps_cur = cps_next

        for g in range(NCHUNK // L):
            b0 = g * L
            rows = [(iota + b0) * NT + t for t in range(NT)]

            def dot_body(d, accs):
                cm = cm_t_v[d, pl.ds(b0, L)]
                cold = jnp.full((L,), d, dtype=jnp.int32)
                return tuple(
                    accs[t] + cm * plsc.load_gather(out_rows_v, [rows[t], cold])
                    for t in range(NT))

            accs = lax.fori_loop(
                0, D, dot_body,
                tuple(jnp.zeros((L,), jnp.float32) for _ in range(NT)))
            for t in range(NT):
                scores_v[t, pl.ds(chunk * NCHUNK + b0, L)] = accs[t]

    for t in range(NT):
        pltpu.sync_copy(scores_v.at[t],
                        scores_hbm.at[t, pl.ds(wid * B_PER_W, B_PER_W)])


@jax.jit
def _sc_scores(ctx_idx, out_idx, in_embed, out_embed):
    mesh = plsc.VectorSubcoreMesh(core_axis_name="c", subcore_axis_name="s")
    f = functools.partial(
        pl.kernel, mesh=mesh,
        out_type=jax.ShapeDtypeStruct((NT, B), jnp.float32),
        scratch_types=[
            pltpu.VMEM((B_PER_W * C,), jnp.int32),
            pltpu.VMEM((B_PER_W * NT,), jnp.int32),
            pltpu.VMEM((NCHUNK * C, D), jnp.float32),
            pltpu.VMEM((NCHUNK * C, D), jnp.float32),
            pltpu.VMEM((NCHUNK * NT, D), jnp.float32),
            pltpu.VMEM((NCHUNK * NT, D), jnp.float32),
            pltpu.VMEM((D, NCHUNK), jnp.float32),
            pltpu.VMEM((NT, B_PER_W), jnp.float32),
            pltpu.SemaphoreType.DMA,
            pltpu.SemaphoreType.DMA,
            pltpu.SemaphoreType.DMA,
            pltpu.SemaphoreType.DMA,
        ],
        compiler_params=pltpu.CompilerParams(
            needs_layout_passes=False, use_tc_tiling_on_sc=False),
    )(_sc_scores_kernel)
    return f(ctx_idx, out_idx, in_embed, out_embed)


def _loss_body(s_ref, o_ref):
    x = s_ref[...]
    is_pos = lax.broadcasted_iota(jnp.int32, x.shape, 0) == 0
    y = jnp.where(is_pos, -x, x)
    sp = jnp.maximum(y, 0.0) + jnp.log(1.0 + jnp.exp(-jnp.abs(y)))
    o_ref[0, 0] = jnp.sum(sp) * (1.0 / B)


@jax.jit
def _tc_loss(scores):
    return pl.pallas_call(
        _loss_body,
        out_shape=jax.ShapeDtypeStruct((1, 1), jnp.float32),
        out_specs=pl.BlockSpec(memory_space=pltpu.SMEM),
    )(scores)


def kernel(center, context, neg_context, in_embed, out_embed):
    ctx_idx = context.astype(jnp.int32).reshape(B * C)
    out_idx = jnp.concatenate(
        [center.astype(jnp.int32), neg_context.astype(jnp.int32)],
        axis=1).reshape(B * NT)
    scores = _sc_scores(ctx_idx, out_idx, in_embed, out_embed)
    loss = _tc_loss(scores)
    return loss[0, 0]
